# Initial kernel scaffold; baseline (speedup 1.0000x reference)
import functools

import jax
import jax.numpy as jnp
from jax import lax
from jax.experimental import pallas as pl
from jax.experimental.pallas import tpu as pltpu
from jax.experimental.pallas import tpu_sc as plsc

N = 10000
E = 160000
H = 128
NC, NS = 2, 16
NW = NC * NS
EPW = E // NW
BATCH = 40
WAVE = 5
ROWS = BATCH * WAVE
NWAVES = EPW // ROWS
NCHUNK = EPW // BATCH
STRIPE = 640
NPAD = NS * STRIPE

f32 = jnp.float32
bf16 = jnp.bfloat16

EB = 2000
NB = 2000


def _dot(x, w):
    return jnp.dot(x.astype(bf16), w, preferred_element_type=f32)


def _tail(pre, w1, b1, w2, b2, w3, b3, g, beta):
    h = jnp.maximum(pre, 0.0)
    h = jnp.maximum(_dot(h, w1[...]) + b1[...], 0.0)
    h = jnp.maximum(_dot(h, w2[...]) + b2[...], 0.0)
    h = _dot(h, w3[...]) + b3[...]
    if g is not None:
        mu = jnp.mean(h, axis=-1, keepdims=True)
        var = jnp.mean((h - mu) ** 2, axis=-1, keepdims=True)
        h = (h - mu) * lax.rsqrt(var + 1e-5) * g[...] + beta[...]
    return h


def _full(shape):
    return pl.BlockSpec(shape, lambda i: (0,) * len(shape))


def _rows(block, width):
    return pl.BlockSpec((block, width), lambda i: (i, 0))


def _prep(p, lay_norm):
    ws = [w.astype(bf16) for w in p['W']]
    bs = [b.reshape(1, -1) for b in p['b']]
    if lay_norm:
        return ws, bs, p['g'].reshape(1, -1), p['beta'].reshape(1, -1)
    return ws, bs, None, None


def _enc_body(x, w0, b0, w1, b1, w2, b2, w3, b3, g, beta, out):
    pre = _dot(x[...], w0[...]) + b0[...]
    out[...] = _tail(pre, w1, b1, w2, b2, w3, b3, g, beta)


def _encoder(x, p, block):
    ws, bs, g, beta = _prep(p, True)
    rows, width = x.shape
    args = [x, ws[0], bs[0], ws[1], bs[1], ws[2], bs[2], ws[3], bs[3], g, beta]
    specs = [_rows(block, width)] + [_full(a.shape) for a in args[1:]]
    return pl.pallas_call(
        _enc_body,
        grid=(rows // block,),
        in_specs=specs,
        out_specs=_rows(block, H),
        out_shape=jax.ShapeDtypeStruct((rows, H), f32),
    )(*args)


def _dec_body(x, w0, b0, w1, b1, w2, b2, w3, b3, out):
    pre = _dot(x[...], w0[...]) + b0[...]
    out[...] = _tail(pre, w1, b1, w2, b2, w3, b3, None, None)


def _decoder(x, p, block):
    ws, bs, _, _ = _prep(p, False)
    rows = x.shape[0]
    out_w = p['W'][3].shape[1]
    args = [x, ws[0], bs[0], ws[1], bs[1], ws[2], bs[2], ws[3], bs[3]]
    specs = [_rows(block, H)] + [_full(a.shape) for a in args[1:]]
    return pl.pallas_call(
        _dec_body,
        grid=(rows // block,),
        in_specs=specs,
        out_specs=_rows(block, out_w),
        out_shape=jax.ShapeDtypeStruct((rows, out_w), f32),
    )(*args)


def _ab_body(nf, w0a, w0b, a_out, b_out):
    nfb = nf[...].astype(bf16)
    a_out[...] = jnp.dot(nfb, w0a[...], preferred_element_type=f32)
    b_out[...] = jnp.dot(nfb, w0b[...], preferred_element_type=f32)


def _ab(nf, w0a, w0b):
    args = [nf, w0a, w0b]
    specs = [_rows(NB, H), _full(w0a.shape), _full(w0b.shape)]
    return pl.pallas_call(
        _ab_body,
        grid=(N // NB,),
        in_specs=specs,
        out_specs=[_rows(NB, H), _rows(NB, H)],
        out_shape=[jax.ShapeDtypeStruct((N, H), f32),
                   jax.ShapeDtypeStruct((N, H), f32)],
    )(*args)


def _edge_body(ga, gb, ef, w0c, b0, w1, b1, w2, b2, w3, b3, g, beta, out):
    efv = ef[...]
    pre = ga[...] + gb[...] + _dot(efv, w0c[...]) + b0[...]
    out[...] = _tail(pre, w1, b1, w2, b2, w3, b3, g, beta) + efv


def _edge_mlp(ga, gb, ef, p):
    ws, bs, g, beta = _prep(p, True)
    w0c = ws[0][2 * H:3 * H]
    args = [ga, gb, ef, w0c, bs[0], ws[1], bs[1], ws[2], bs[2], ws[3], bs[3],
            g, beta]
    specs = [_rows(EB, H)] * 3 + [_full(a.shape) for a in args[3:]]
    return pl.pallas_call(
        _edge_body,
        grid=(E // EB,),
        in_specs=specs,
        out_specs=_rows(EB, H),
        out_shape=jax.ShapeDtypeStruct((E, H), f32),
    )(*args)


def _node_body(nf, a0, a1, w0a, w0b, b0, w1, b1, w2, b2, w3, b3, g, beta, out):
    nfv = nf[...]
    agg = (a0[...] + a1[...]).astype(bf16)
    pre = (_dot(nfv, w0a[...]) +
           jnp.dot(agg, w0b[...], preferred_element_type=f32) + b0[...])
    out[...] = _tail(pre, w1, b1, w2, b2, w3, b3, g, beta) + nfv


def _node_mlp(nf, a0, a1, p):
    ws, bs, g, beta = _prep(p, True)
    w0a, w0b = ws[0][:H], ws[0][H:2 * H]
    args = [nf, a0, a1, w0a, w0b, bs[0], ws[1], bs[1], ws[2], bs[2], ws[3],
            bs[3], g, beta]
    specs = [_rows(NB, H)] * 3 + [_full(a.shape) for a in args[3:]]
    return pl.pallas_call(
        _node_body,
        grid=(N // NB,),
        in_specs=specs,
        out_specs=_rows(NB, H),
        out_shape=jax.ShapeDtypeStruct((N, H), f32),
    )(*args)


_MESH = plsc.VectorSubcoreMesh(core_axis_name="c", subcore_axis_name="s")


@functools.partial(
    pl.kernel,
    out_type=(jax.ShapeDtypeStruct((E, H), f32),
              jax.ShapeDtypeStruct((E, H), f32)),
    mesh=_MESH,
    scratch_types=[
        pltpu.VMEM((EPW,), jnp.int32),
        pltpu.VMEM((EPW,), jnp.int32),
        pltpu.VMEM((ROWS, H), f32),
        pltpu.VMEM((ROWS, H), f32),
        pltpu.SemaphoreType.DMA,
        pltpu.SemaphoreType.DMA,
    ],
)
def _gather_sc(a_hbm, b_hbm, s_hbm, r_hbm, ga_hbm, gb_hbm,
               sidx, ridx, abuf, bbuf, sem_g, sem_w):
    cid = lax.axis_index("c")
    sid = lax.axis_index("s")
    wid = sid * NC + cid
    base = pl.multiple_of(wid * EPW, ROWS)
    pltpu.sync_copy(s_hbm.at[pl.ds(base, EPW)], sidx)
    pltpu.sync_copy(r_hbm.at[pl.ds(base, EPW)], ridx)

    def wave(w, carry):
        wbase = pl.multiple_of(w * ROWS, ROWS)
        handles = []
        for j in range(WAVE):
            o = pl.multiple_of(wbase + j * BATCH, BATCH)
            handles.append(pltpu.async_copy(
                a_hbm.at[sidx.at[pl.ds(o, BATCH)]],
                abuf.at[pl.ds(j * BATCH, BATCH)], sem_g))
            handles.append(pltpu.async_copy(
                b_hbm.at[ridx.at[pl.ds(o, BATCH)]],
                bbuf.at[pl.ds(j * BATCH, BATCH)], sem_g))
        for hd in handles:
            hd.wait()
        out_off = pl.multiple_of(base + wbase, ROWS)
        wa = pltpu.async_copy(abuf, ga_hbm.at[pl.ds(out_off, ROWS)], sem_w)
        wb = pltpu.async_copy(bbuf, gb_hbm.at[pl.ds(out_off, ROWS)], sem_w)
        wa.wait()
        wb.wait()
        return carry

    lax.fori_loop(0, NWAVES, wave, 0)


@functools.partial(
    pl.kernel,
    out_type=jax.ShapeDtypeStruct((NC, NPAD, H), f32),
    mesh=_MESH,
    scratch_types=[
        pltpu.VMEM((NCHUNK, BATCH), jnp.int32),
        pltpu.VMEM((ROWS, H), f32),
        pltpu.VMEM_SHARED((NPAD, H), f32),
        pltpu.SemaphoreType.DMA,
    ],
)
def _scatter_sc(vals_hbm, ridx3_hbm, zeros_hbm, out_hbm, idxv, vbuf, acc, sem):
    cid = lax.axis_index("c")
    sid = lax.axis_index("s")
    wid = sid * NC + cid
    base = pl.multiple_of(wid * EPW, ROWS)
    pltpu.sync_copy(ridx3_hbm.at[wid], idxv)
    stripe = pl.multiple_of(sid * STRIPE, STRIPE)
    pltpu.sync_copy(zeros_hbm, acc.at[pl.ds(stripe, STRIPE)])
    plsc.subcore_barrier()

    def wave(w, carry):
        voff = pl.multiple_of(base + w * ROWS, ROWS)
        pltpu.sync_copy(vals_hbm.at[pl.ds(voff, ROWS)], vbuf)
        handles = []
        for j in range(WAVE):
            handles.append(pltpu.async_copy(
                vbuf.at[pl.ds(j * BATCH, BATCH)],
                acc.at[idxv.at[w * WAVE + j]], sem, add=True))
        for hd in handles:
            hd.wait()
        return carry

    lax.fori_loop(0, NWAVES, wave, 0)
    plsc.subcore_barrier()
    pltpu.sync_copy(acc.at[pl.ds(stripe, STRIPE)],
                    out_hbm.at[cid, pl.ds(stripe, STRIPE)])


def kernel(node_attr, edge_attr, edge_index, params):
    s = edge_index[0].astype(jnp.int32)
    r = edge_index[1].astype(jnp.int32)
    ridx3 = r.reshape(NW, NCHUNK, BATCH)
    zeros = jnp.zeros((STRIPE, H), f32)

    nf = _encoder(node_attr, params['node_encoder'], NB)
    ef = _encoder(edge_attr, params['edge_encoder'], EB)

    for blk in params['blocks']:
        pE, pN = blk['edge_mlp'], blk['node_mlp']
        w0 = pE['W'][0].astype(bf16)
        a_tab, b_tab = _ab(nf, w0[:H], w0[H:2 * H])
        ga, gb = _gather_sc(a_tab, b_tab, s, r)
        e_new = _edge_mlp(ga, gb, ef, pE)
        parts = _scatter_sc(e_new, ridx3, zeros)
        nf = _node_mlp(nf, parts[0, :N], parts[1, :N], pN)
        ef = e_new
    return _decoder(nf, params['decoder'], NB)


# speedup vs baseline: 3.3040x; 3.3040x over previous
import functools

import jax
import jax.numpy as jnp
from jax import lax
from jax.experimental import pallas as pl
from jax.experimental.pallas import tpu as pltpu
from jax.experimental.pallas import tpu_sc as plsc

N = 10000
E = 160000
H = 128
NC, NS = 2, 16
NW = NC * NS
EPW = E // NW
BATCH = 40
WAVE = 5
ROWS = BATCH * WAVE
NWAVES = EPW // ROWS
NCHUNK = EPW // BATCH
STRIPE = 640
NPAD = NS * STRIPE

f32 = jnp.float32
bf16 = jnp.bfloat16

EB = 2000
NB = 2000


def _dot(x, w):
    return jnp.dot(x.astype(bf16), w, preferred_element_type=f32)


def _tail(pre, w1, b1, w2, b2, w3, b3, g, beta):
    h = jnp.maximum(pre, 0.0)
    h = jnp.maximum(_dot(h, w1[...]) + b1[...], 0.0)
    h = jnp.maximum(_dot(h, w2[...]) + b2[...], 0.0)
    h = _dot(h, w3[...]) + b3[...]
    if g is not None:
        mu = jnp.mean(h, axis=-1, keepdims=True)
        var = jnp.mean((h - mu) ** 2, axis=-1, keepdims=True)
        h = (h - mu) * lax.rsqrt(var + 1e-5) * g[...] + beta[...]
    return h


def _full(shape):
    return pl.BlockSpec(shape, lambda i: (0,) * len(shape))


def _rows(block, width):
    return pl.BlockSpec((block, width), lambda i: (i, 0))


def _prep(p, lay_norm):
    ws = [w.astype(bf16) for w in p['W']]
    bs = [b.reshape(1, -1) for b in p['b']]
    if lay_norm:
        return ws, bs, p['g'].reshape(1, -1), p['beta'].reshape(1, -1)
    return ws, bs, None, None


def _enc_body(x, w0, b0, w1, b1, w2, b2, w3, b3, g, beta, out):
    pre = _dot(x[...], w0[...]) + b0[...]
    out[...] = _tail(pre, w1, b1, w2, b2, w3, b3, g, beta)


def _encoder(x, p, block):
    ws, bs, g, beta = _prep(p, True)
    rows, width = x.shape
    args = [x, ws[0], bs[0], ws[1], bs[1], ws[2], bs[2], ws[3], bs[3], g, beta]
    specs = [_rows(block, width)] + [_full(a.shape) for a in args[1:]]
    return pl.pallas_call(
        _enc_body,
        grid=(rows // block,),
        in_specs=specs,
        out_specs=_rows(block, H),
        out_shape=jax.ShapeDtypeStruct((rows, H), f32),
    )(*args)


def _dec_body(x, w0, b0, w1, b1, w2, b2, w3, b3, out):
    pre = _dot(x[...], w0[...]) + b0[...]
    out[...] = _tail(pre, w1, b1, w2, b2, w3, b3, None, None)


def _decoder(x, p, block):
    ws, bs, _, _ = _prep(p, False)
    rows = x.shape[0]
    out_w = p['W'][3].shape[1]
    args = [x, ws[0], bs[0], ws[1], bs[1], ws[2], bs[2], ws[3], bs[3]]
    specs = [_rows(block, H)] + [_full(a.shape) for a in args[1:]]
    return pl.pallas_call(
        _dec_body,
        grid=(rows // block,),
        in_specs=specs,
        out_specs=_rows(block, out_w),
        out_shape=jax.ShapeDtypeStruct((rows, out_w), f32),
    )(*args)


def _ab_body(nf, w0a, w0b, a_out, b_out):
    nfb = nf[...].astype(bf16)
    a_out[...] = jnp.dot(nfb, w0a[...], preferred_element_type=f32)
    b_out[...] = jnp.dot(nfb, w0b[...], preferred_element_type=f32)


def _ab(nf, w0a, w0b):
    args = [nf, w0a, w0b]
    specs = [_rows(NB, H), _full(w0a.shape), _full(w0b.shape)]
    return pl.pallas_call(
        _ab_body,
        grid=(N // NB,),
        in_specs=specs,
        out_specs=[_rows(NB, H), _rows(NB, H)],
        out_shape=[jax.ShapeDtypeStruct((N, H), f32),
                   jax.ShapeDtypeStruct((N, H), f32)],
    )(*args)


def _edge_body(ga, gb, ef, w0c, b0, w1, b1, w2, b2, w3, b3, g, beta, out):
    efv = ef[...]
    pre = ga[...] + gb[...] + _dot(efv, w0c[...]) + b0[...]
    out[...] = _tail(pre, w1, b1, w2, b2, w3, b3, g, beta) + efv


def _edge_mlp(ga, gb, ef, p):
    ws, bs, g, beta = _prep(p, True)
    w0c = ws[0][2 * H:3 * H]
    args = [ga, gb, ef, w0c, bs[0], ws[1], bs[1], ws[2], bs[2], ws[3], bs[3],
            g, beta]
    specs = [_rows(EB, H)] * 3 + [_full(a.shape) for a in args[3:]]
    return pl.pallas_call(
        _edge_body,
        grid=(E // EB,),
        in_specs=specs,
        out_specs=_rows(EB, H),
        out_shape=jax.ShapeDtypeStruct((E, H), f32),
    )(*args)


def _node_body(nf, a0, a1, w0a, w0b, b0, w1, b1, w2, b2, w3, b3, g, beta, out):
    nfv = nf[...]
    agg = (a0[...] + a1[...]).astype(bf16)
    pre = (_dot(nfv, w0a[...]) +
           jnp.dot(agg, w0b[...], preferred_element_type=f32) + b0[...])
    out[...] = _tail(pre, w1, b1, w2, b2, w3, b3, g, beta) + nfv


def _node_mlp(nf, a0, a1, p):
    ws, bs, g, beta = _prep(p, True)
    w0a, w0b = ws[0][:H], ws[0][H:2 * H]
    args = [nf, a0, a1, w0a, w0b, bs[0], ws[1], bs[1], ws[2], bs[2], ws[3],
            bs[3], g, beta]
    specs = [_rows(NB, H)] * 3 + [_full(a.shape) for a in args[3:]]
    return pl.pallas_call(
        _node_body,
        grid=(N // NB,),
        in_specs=specs,
        out_specs=_rows(NB, H),
        out_shape=jax.ShapeDtypeStruct((N, H), f32),
    )(*args)


@functools.cache
def _gather_sc_build():
    mesh = plsc.VectorSubcoreMesh(core_axis_name="c", subcore_axis_name="s",
                                  num_cores=NC, num_subcores=NS)
    return functools.partial(
        pl.kernel,
        out_type=(jax.ShapeDtypeStruct((E, H), f32),
                  jax.ShapeDtypeStruct((E, H), f32)),
        mesh=mesh,
        scratch_types=[
            pltpu.VMEM((EPW,), jnp.int32),
            pltpu.VMEM((EPW,), jnp.int32),
            pltpu.VMEM((ROWS, H), f32),
            pltpu.VMEM((ROWS, H), f32),
            pltpu.SemaphoreType.DMA,
            pltpu.SemaphoreType.DMA,
        ],
    )(_gather_sc_body)


def _gather_sc(a_tab, b_tab, s, r):
    return _gather_sc_build()(a_tab, b_tab, s, r)


def _gather_sc_body(a_hbm, b_hbm, s_hbm, r_hbm, ga_hbm, gb_hbm,
                    sidx, ridx, abuf, bbuf, sem_g, sem_w):
    cid = lax.axis_index("c")
    sid = lax.axis_index("s")
    wid = sid * NC + cid
    base = pl.multiple_of(wid * EPW, ROWS)
    pltpu.sync_copy(s_hbm.at[pl.ds(base, EPW)], sidx)
    pltpu.sync_copy(r_hbm.at[pl.ds(base, EPW)], ridx)

    def wave(w, carry):
        wbase = pl.multiple_of(w * ROWS, ROWS)
        handles = []
        for j in range(WAVE):
            o = pl.multiple_of(wbase + j * BATCH, BATCH)
            handles.append(pltpu.async_copy(
                a_hbm.at[sidx.at[pl.ds(o, BATCH)]],
                abuf.at[pl.ds(j * BATCH, BATCH)], sem_g))
            handles.append(pltpu.async_copy(
                b_hbm.at[ridx.at[pl.ds(o, BATCH)]],
                bbuf.at[pl.ds(j * BATCH, BATCH)], sem_g))
        for hd in handles:
            hd.wait()
        out_off = pl.multiple_of(base + wbase, ROWS)
        wa = pltpu.async_copy(abuf, ga_hbm.at[pl.ds(out_off, ROWS)], sem_w)
        wb = pltpu.async_copy(bbuf, gb_hbm.at[pl.ds(out_off, ROWS)], sem_w)
        wa.wait()
        wb.wait()
        return carry

    lax.fori_loop(0, NWAVES, wave, 0)


@functools.cache
def _scatter_sc_build():
    mesh = plsc.VectorSubcoreMesh(core_axis_name="c", subcore_axis_name="s",
                                  num_cores=NC, num_subcores=NS)
    return functools.partial(
        pl.kernel,
        out_type=jax.ShapeDtypeStruct((NC, NPAD, H), f32),
        mesh=mesh,
        scratch_types=[
            pltpu.VMEM((NCHUNK, BATCH), jnp.int32),
            pltpu.VMEM((ROWS, H), f32),
            pltpu.VMEM_SHARED((NPAD, H), f32),
            pltpu.SemaphoreType.DMA,
        ],
    )(_scatter_sc_body)


def _scatter_sc(e_new, ridx3, zeros):
    return _scatter_sc_build()(e_new, ridx3, zeros)


def _scatter_sc_body(vals_hbm, ridx3_hbm, zeros_hbm, out_hbm,
                     idxv, vbuf, acc, sem):
    cid = lax.axis_index("c")
    sid = lax.axis_index("s")
    wid = sid * NC + cid
    base = pl.multiple_of(wid * EPW, ROWS)
    pltpu.sync_copy(ridx3_hbm.at[wid], idxv)
    stripe = pl.multiple_of(sid * STRIPE, STRIPE)
    pltpu.sync_copy(zeros_hbm, acc.at[pl.ds(stripe, STRIPE)])
    plsc.subcore_barrier()

    def wave(w, carry):
        voff = pl.multiple_of(base + w * ROWS, ROWS)
        pltpu.sync_copy(vals_hbm.at[pl.ds(voff, ROWS)], vbuf)
        handles = []
        for j in range(WAVE):
            handles.append(pltpu.async_copy(
                vbuf.at[pl.ds(j * BATCH, BATCH)],
                acc.at[idxv.at[w * WAVE + j]], sem, add=True))
        for hd in handles:
            hd.wait()
        return carry

    lax.fori_loop(0, NWAVES, wave, 0)
    plsc.subcore_barrier()
    pltpu.sync_copy(acc.at[pl.ds(stripe, STRIPE)],
                    out_hbm.at[cid, pl.ds(stripe, STRIPE)])


def kernel(node_attr, edge_attr, edge_index, params):
    s = edge_index[0].astype(jnp.int32)
    r = edge_index[1].astype(jnp.int32)
    ridx3 = r.reshape(NW, NCHUNK, BATCH)
    zeros = jnp.zeros((STRIPE, H), f32)

    nf = _encoder(node_attr, params['node_encoder'], NB)
    ef = _encoder(edge_attr, params['edge_encoder'], EB)

    for blk in params['blocks']:
        pE, pN = blk['edge_mlp'], blk['node_mlp']
        w0 = pE['W'][0].astype(bf16)
        a_tab, b_tab = _ab(nf, w0[:H], w0[H:2 * H])
        ga, gb = _gather_sc(a_tab, b_tab, s, r)
        e_new = _edge_mlp(ga, gb, ef, pE)
        parts = _scatter_sc(e_new, ridx3, zeros)
        nf = _node_mlp(nf, parts[0, :N], parts[1, :N], pN)
        ef = e_new
    return _decoder(nf, params['decoder'], NB)


# speedup vs baseline: 3.3207x; 1.0051x over previous
import functools

import jax
import jax.numpy as jnp
from jax import lax
from jax.experimental import pallas as pl
from jax.experimental.pallas import tpu as pltpu
from jax.experimental.pallas import tpu_sc as plsc

N = 10000
E = 160000
H = 128
NC, NS = 2, 16
NW = NC * NS
EPW = E // NW
BATCH = 40
WAVE = 5
ROWS = BATCH * WAVE
NWAVES = EPW // ROWS
NCHUNK = EPW // BATCH
STRIPE = 640
NPAD = NS * STRIPE

f32 = jnp.float32
bf16 = jnp.bfloat16

EB = 2000
NB = 2000


def _dot(x, w):
    return jnp.dot(x.astype(bf16), w, preferred_element_type=f32)


def _tail(pre, w1, b1, w2, b2, w3, b3, g, beta):
    h = jnp.maximum(pre, 0.0)
    h = jnp.maximum(_dot(h, w1[...]) + b1[...], 0.0)
    h = jnp.maximum(_dot(h, w2[...]) + b2[...], 0.0)
    h = _dot(h, w3[...]) + b3[...]
    if g is not None:
        mu = jnp.mean(h, axis=-1, keepdims=True)
        var = jnp.mean((h - mu) ** 2, axis=-1, keepdims=True)
        h = (h - mu) * lax.rsqrt(var + 1e-5) * g[...] + beta[...]
    return h


def _full(shape):
    return pl.BlockSpec(shape, lambda i: (0,) * len(shape))


def _rows(block, width):
    return pl.BlockSpec((block, width), lambda i: (i, 0))


def _prep(p, lay_norm):
    ws = [w.astype(bf16) for w in p['W']]
    bs = [b.reshape(1, -1) for b in p['b']]
    if lay_norm:
        return ws, bs, p['g'].reshape(1, -1), p['beta'].reshape(1, -1)
    return ws, bs, None, None


def _enc_body(x, w0, b0, w1, b1, w2, b2, w3, b3, g, beta, out):
    pre = _dot(x[...], w0[...]) + b0[...]
    out[...] = _tail(pre, w1, b1, w2, b2, w3, b3, g, beta)


def _encoder(x, p, block):
    ws, bs, g, beta = _prep(p, True)
    rows, width = x.shape
    args = [x, ws[0], bs[0], ws[1], bs[1], ws[2], bs[2], ws[3], bs[3], g, beta]
    specs = [_rows(block, width)] + [_full(a.shape) for a in args[1:]]
    return pl.pallas_call(
        _enc_body,
        grid=(rows // block,),
        in_specs=specs,
        out_specs=_rows(block, H),
        out_shape=jax.ShapeDtypeStruct((rows, H), f32),
    )(*args)


def _dec_body(x, w0, b0, w1, b1, w2, b2, w3, b3, out):
    pre = _dot(x[...], w0[...]) + b0[...]
    out[...] = _tail(pre, w1, b1, w2, b2, w3, b3, None, None)


def _decoder(x, p, block):
    ws, bs, _, _ = _prep(p, False)
    rows = x.shape[0]
    out_w = p['W'][3].shape[1]
    args = [x, ws[0], bs[0], ws[1], bs[1], ws[2], bs[2], ws[3], bs[3]]
    specs = [_rows(block, H)] + [_full(a.shape) for a in args[1:]]
    return pl.pallas_call(
        _dec_body,
        grid=(rows // block,),
        in_specs=specs,
        out_specs=_rows(block, out_w),
        out_shape=jax.ShapeDtypeStruct((rows, out_w), f32),
    )(*args)


def _ab_body(nf, w0a, w0b, a_out, b_out):
    nfb = nf[...].astype(bf16)
    a_out[...] = jnp.dot(nfb, w0a[...], preferred_element_type=f32)
    b_out[...] = jnp.dot(nfb, w0b[...], preferred_element_type=f32)


def _ab(nf, w0a, w0b):
    args = [nf, w0a, w0b]
    specs = [_rows(NB, H), _full(w0a.shape), _full(w0b.shape)]
    return pl.pallas_call(
        _ab_body,
        grid=(N // NB,),
        in_specs=specs,
        out_specs=[_rows(NB, H), _rows(NB, H)],
        out_shape=[jax.ShapeDtypeStruct((N, H), f32),
                   jax.ShapeDtypeStruct((N, H), f32)],
    )(*args)


def _edge_body(gsum, ef, w0c, b0, w1, b1, w2, b2, w3, b3, g, beta, out):
    efv = ef[...]
    pre = gsum[...] + _dot(efv, w0c[...]) + b0[...]
    out[...] = _tail(pre, w1, b1, w2, b2, w3, b3, g, beta) + efv


def _edge_mlp(gsum, ef, p):
    ws, bs, g, beta = _prep(p, True)
    w0c = ws[0][2 * H:3 * H]
    args = [gsum, ef, w0c, bs[0], ws[1], bs[1], ws[2], bs[2], ws[3], bs[3],
            g, beta]
    specs = [_rows(EB, H)] * 2 + [_full(a.shape) for a in args[2:]]
    return pl.pallas_call(
        _edge_body,
        grid=(E // EB,),
        in_specs=specs,
        out_specs=_rows(EB, H),
        out_shape=jax.ShapeDtypeStruct((E, H), f32),
    )(*args)


def _node_body(nf, a0, a1, w0a, w0b, b0, w1, b1, w2, b2, w3, b3, g, beta, out):
    nfv = nf[...]
    agg = (a0[...] + a1[...]).astype(bf16)
    pre = (_dot(nfv, w0a[...]) +
           jnp.dot(agg, w0b[...], preferred_element_type=f32) + b0[...])
    out[...] = _tail(pre, w1, b1, w2, b2, w3, b3, g, beta) + nfv


def _node_mlp(nf, a0, a1, p):
    ws, bs, g, beta = _prep(p, True)
    w0a, w0b = ws[0][:H], ws[0][H:2 * H]
    args = [nf, a0, a1, w0a, w0b, bs[0], ws[1], bs[1], ws[2], bs[2], ws[3],
            bs[3], g, beta]
    specs = [_rows(NB, H)] * 3 + [_full(a.shape) for a in args[3:]]
    return pl.pallas_call(
        _node_body,
        grid=(N // NB,),
        in_specs=specs,
        out_specs=_rows(NB, H),
        out_shape=jax.ShapeDtypeStruct((N, H), f32),
    )(*args)


@functools.cache
def _gather_sc_build():
    mesh = plsc.VectorSubcoreMesh(core_axis_name="c", subcore_axis_name="s",
                                  num_cores=NC, num_subcores=NS)
    return functools.partial(
        pl.kernel,
        out_type=jax.ShapeDtypeStruct((E, H), f32),
        mesh=mesh,
        scratch_types=[
            pltpu.VMEM((EPW,), jnp.int32),
            pltpu.VMEM((EPW,), jnp.int32),
            pltpu.VMEM((ROWS, H), f32),
            pltpu.VMEM((ROWS, H), f32),
            pltpu.VMEM((ROWS, H), f32),
            pltpu.VMEM((ROWS, H), f32),
            pltpu.SemaphoreType.DMA,
            pltpu.SemaphoreType.DMA,
            pltpu.SemaphoreType.DMA,
        ],
    )(_gather_sc_body)


def _gather_sc(a_tab, b_tab, s, r):
    return _gather_sc_build()(a_tab, b_tab, s, r)


def _gather_sc_body(a_hbm, b_hbm, s_hbm, r_hbm, g_hbm,
                    sidx, ridx, a0, b0, a1, b1, sem_g0, sem_g1, sem_w):
    cid = lax.axis_index("c")
    sid = lax.axis_index("s")
    wid = sid * NC + cid
    base = pl.multiple_of(wid * EPW, ROWS)
    pltpu.sync_copy(s_hbm.at[pl.ds(base, EPW)], sidx)
    pltpu.sync_copy(r_hbm.at[pl.ds(base, EPW)], ridx)

    def fire(w, abuf, bbuf, sem):
        wbase = pl.multiple_of(w * ROWS, 8)
        hs = []
        for j in range(WAVE):
            o = pl.multiple_of(wbase + j * BATCH, 8)
            hs.append(pltpu.async_copy(
                a_hbm.at[sidx.at[pl.ds(o, BATCH)]],
                abuf.at[pl.ds(j * BATCH, BATCH)], sem))
            hs.append(pltpu.async_copy(
                b_hbm.at[ridx.at[pl.ds(o, BATCH)]],
                bbuf.at[pl.ds(j * BATCH, BATCH)], sem))
        return hs

    def add_into(abuf, bbuf):
        def row(i, c):
            for j in range(H // 16):
                sl = pl.ds(j * 16, 16)
                abuf[i, sl] = abuf[i, sl] + bbuf[i, sl]
            return c
        lax.fori_loop(0, ROWS, row, 0)

    def writeback(w, abuf):
        off = pl.multiple_of(base + w * ROWS, 8)
        return pltpu.async_copy(abuf, g_hbm.at[pl.ds(off, ROWS)], sem_w)

    def pair(k, carry):
        w = 2 * k
        h0 = fire(w, a0, b0, sem_g0)
        h1 = fire(w + 1, a1, b1, sem_g1)
        for hd in h0:
            hd.wait()
        add_into(a0, b0)
        wb0 = writeback(w, a0)
        for hd in h1:
            hd.wait()
        add_into(a1, b1)
        wb1 = writeback(w + 1, a1)
        wb0.wait()
        wb1.wait()
        return carry

    lax.fori_loop(0, NWAVES // 2, pair, 0)
    if NWAVES % 2:
        w = NWAVES - 1
        for hd in fire(w, a0, b0, sem_g0):
            hd.wait()
        add_into(a0, b0)
        writeback(w, a0).wait()


@functools.cache
def _scatter_sc_build():
    mesh = plsc.VectorSubcoreMesh(core_axis_name="c", subcore_axis_name="s",
                                  num_cores=NC, num_subcores=NS)
    return functools.partial(
        pl.kernel,
        out_type=jax.ShapeDtypeStruct((NC, NPAD, H), f32),
        mesh=mesh,
        scratch_types=[
            pltpu.VMEM((NCHUNK, BATCH), jnp.int32),
            pltpu.VMEM((BATCH, H), f32),
            pltpu.VMEM((BATCH, H), f32),
            pltpu.VMEM_SHARED((NPAD, H), f32),
            pltpu.SemaphoreType.DMA,
            pltpu.SemaphoreType.DMA,
            pltpu.SemaphoreType.DMA,
        ],
    )(_scatter_sc_body)


def _scatter_sc(e_new, ridx3, zeros):
    return _scatter_sc_build()(e_new, ridx3, zeros)


def _scatter_sc_body(vals_hbm, ridx3_hbm, zeros_hbm, out_hbm,
                     idxv, v0, v1, acc, sem_s0, sem_s1, sem_a):
    cid = lax.axis_index("c")
    sid = lax.axis_index("s")
    wid = sid * NC + cid
    base = pl.multiple_of(wid * EPW, 8)
    pltpu.sync_copy(ridx3_hbm.at[wid], idxv)
    stripe = pl.multiple_of(sid * STRIPE, 8)
    pltpu.sync_copy(zeros_hbm, acc.at[pl.ds(stripe, STRIPE)])
    plsc.subcore_barrier()

    def stage(w, vbuf, sem):
        voff = pl.multiple_of(base + w * BATCH, 8)
        return pltpu.async_copy(vals_hbm.at[pl.ds(voff, BATCH)], vbuf, sem)

    def scat(w, vbuf):
        return pltpu.async_copy(vbuf, acc.at[idxv.at[w]], sem_a, add=True)

    def pair(k, carry):
        w = 2 * k
        st0 = stage(w, v0, sem_s0)
        st1 = stage(w + 1, v1, sem_s1)
        st0.wait()
        h0 = scat(w, v0)
        st1.wait()
        h1 = scat(w + 1, v1)
        h0.wait()
        h1.wait()
        return carry

    lax.fori_loop(0, NCHUNK // 2, pair, 0)
    if NCHUNK % 2:
        w = NCHUNK - 1
        stage(w, v0, sem_s0).wait()
        scat(w, v0).wait()
    plsc.subcore_barrier()
    pltpu.sync_copy(acc.at[pl.ds(stripe, STRIPE)],
                    out_hbm.at[cid, pl.ds(stripe, STRIPE)])


def kernel(node_attr, edge_attr, edge_index, params):
    s = edge_index[0].astype(jnp.int32)
    r = edge_index[1].astype(jnp.int32)
    ridx3 = r.reshape(NW, NCHUNK, BATCH)
    zeros = jnp.zeros((STRIPE, H), f32)

    nf = _encoder(node_attr, params['node_encoder'], NB)
    ef = _encoder(edge_attr, params['edge_encoder'], EB)

    for blk in params['blocks']:
        pE, pN = blk['edge_mlp'], blk['node_mlp']
        w0 = pE['W'][0].astype(bf16)
        a_tab, b_tab = _ab(nf, w0[:H], w0[H:2 * H])
        gsum = _gather_sc(a_tab, b_tab, s, r)
        e_new = _edge_mlp(gsum, ef, pE)
        parts = _scatter_sc(e_new, ridx3, zeros)
        nf = _node_mlp(nf, parts[0, :N], parts[1, :N], pN)
        ef = e_new
    return _decoder(nf, params['decoder'], NB)
